# baseline (speedup 1.0000x reference)
import functools
import jax
import jax.numpy as jnp
from jax import lax
from jax.experimental import pallas as pl
from jax.experimental.pallas import tpu as pltpu
from jax.experimental.pallas import tpu_sc as plsc

DW = 16
PK = 128 // DW
NW = 32
NT = 6


def _gather_kernel(n, chunk):
    mesh = plsc.VectorSubcoreMesh(core_axis_name="c", subcore_axis_name="s")
    tpw = n // NW
    g_steps = tpw // chunk
    pc = chunk // PK
    lines_t = n // PK

    @functools.partial(
        pl.kernel,
        mesh=mesh,
        out_type=jax.ShapeDtypeStruct((NT * lines_t, 128), jnp.float32),
        scratch_types=[
            pltpu.VMEM((2, chunk), jnp.int32),
            pltpu.VMEM((2, PK, pc, DW), jnp.float32),
            pltpu.SemaphoreType.DMA,
            pltpu.SemaphoreType.DMA,
            pltpu.SemaphoreType.DMA,
            pltpu.SemaphoreType.DMA,
        ],
        compiler_params=pltpu.CompilerParams(use_tc_tiling_on_sc=False),
    )
    def gather_k(table_hbm, idx_hbm, out_hbm, idx_v, rows_v, g0, g1, w0, w1):
        wid = lax.axis_index("s") * 2 + lax.axis_index("c")
        gsem = [g0, g1]
        wsem = [w0, w1]
        chunks = [(t, g) for t in range(NT) for g in range(g_steps)]

        def load_idx(c, s):
            t, g = chunks[c]
            off = t * n + wid * tpw + g * chunk
            pltpu.sync_copy(idx_hbm.at[pl.ds(off, chunk)], idx_v.at[s])

        def start_gathers(s):
            return [
                pltpu.async_copy(
                    table_hbm.at[idx_v.at[s, pl.ds(k * pc, pc)]],
                    rows_v.at[s, k],
                    gsem[s])
                for k in range(PK)
            ]

        def start_wb(c, s):
            t, g = chunks[c]
            line0 = t * lines_t + wid * (tpw // PK) + g * pc
            return [
                pltpu.async_copy(
                    rows_v.at[s, k],
                    out_hbm.at[pl.ds(line0, pc), pl.ds(k * DW, DW)],
                    wsem[s])
                for k in range(PK)
            ]

        n_chunks = len(chunks)
        load_idx(0, 0)
        g_h = [start_gathers(0), None]
        w_h = [None, None]
        for c in range(n_chunks):
            s = c & 1
            o = s ^ 1
            if c + 1 < n_chunks:
                if w_h[o] is not None:
                    for h in w_h[o]:
                        h.wait()
                load_idx(c + 1, o)
                g_h[o] = start_gathers(o)
            for h in g_h[s]:
                h.wait()
            w_h[s] = start_wb(c, s)
        for hs in w_h:
            if hs is not None:
                for h in hs:
                    h.wait()

    return gather_k


def _make_fuse_body(pc):
    def _fuse_body(x_ref, w2_ref, b2_ref, gam_ref, bet_ref, o_ref):
        w2 = w2_ref[...]
        b2 = b2_ref[...]
        for k in range(PK):
            xk = jnp.concatenate(
                [x_ref[t][:, k * DW:(k + 1) * DW] for t in range(NT)],
                axis=1)
            lin = jnp.dot(xk, w2, preferred_element_type=jnp.float32) + b2
            gate = jax.nn.sigmoid(lin[:, :128] * 1.2)
            z = gate * lin[:, 128:]
            mu = jnp.mean(z, axis=-1, keepdims=True)
            zc = z - mu
            var = jnp.mean(zc * zc, axis=-1, keepdims=True)
            z_norm = zc * lax.rsqrt(var + 1e-5)
            o_ref[pl.ds(k * pc, pc), :] = z_norm * gam_ref[...] + bet_ref[...]
    return _fuse_body


def kernel(event_type, fault_class, syscall_class, opcode_family,
           transition_type, result_class,
           W_event, W_fault, W_syscall, W_opcode, W_trans, W_result,
           gate_W, gate_b, trans_W, trans_b, ln_gamma, ln_beta):
    tables = [W_event, W_fault, W_syscall, W_opcode, W_trans, W_result]
    idxs = [event_type, fault_class, syscall_class, opcode_family,
            transition_type, result_class]
    widths = [t.shape[1] for t in tables]

    n = event_type.size
    chunk = n // NW // 2
    pc = chunk // PK

    padded, shifted, row_base = [], [], 0
    for W, ix in zip(tables, idxs):
        padded.append(jnp.pad(W, ((0, 0), (0, DW - W.shape[1]))))
        shifted.append(ix.reshape(-1).astype(jnp.int32) + row_base)
        row_base += W.shape[0]
    big_table = jnp.concatenate(padded, axis=0)
    idx_all = jnp.concatenate(shifted)

    gathered = _gather_kernel(n, chunk)(big_table, idx_all)
    gathered = gathered.reshape(NT, n // PK, 128)

    def seg_w(W):
        out, r = [], 0
        for w in widths:
            out.append(jnp.pad(W[r:r + w], ((0, DW - w), (0, 0))))
            r += w
        return jnp.concatenate(out, axis=0)

    w2 = jnp.concatenate([seg_w(gate_W), seg_w(trans_W)], axis=1)
    b2 = jnp.concatenate([gate_b, trans_b]).reshape(1, 256)

    out = pl.pallas_call(
        _make_fuse_body(pc),
        grid=(n // chunk,),
        in_specs=[
            pl.BlockSpec((NT, pc, 128), lambda i: (0, i, 0)),
            pl.BlockSpec((NT * DW, 256), lambda i: (0, 0)),
            pl.BlockSpec((1, 256), lambda i: (0, 0)),
            pl.BlockSpec((1, 128), lambda i: (0, 0)),
            pl.BlockSpec((1, 128), lambda i: (0, 0)),
        ],
        out_specs=pl.BlockSpec((chunk, 128), lambda i: (i, 0)),
        out_shape=jax.ShapeDtypeStruct((n, 128), jnp.float32),
    )(gathered, w2, b2, ln_gamma.reshape(1, 128), ln_beta.reshape(1, 128))

    return out.reshape(event_type.shape + (128,))


# speedup vs baseline: 1.0550x; 1.0550x over previous
import functools
import jax
import jax.numpy as jnp
from jax import lax
from jax.experimental import pallas as pl
from jax.experimental.pallas import tpu as pltpu
from jax.experimental.pallas import tpu_sc as plsc

DW = 8
PK = 128 // DW
NW = 32
NT = 6


def _gather_kernel(n, chunk):
    mesh = plsc.VectorSubcoreMesh(core_axis_name="c", subcore_axis_name="s")
    tpw = n // NW
    g_steps = tpw // chunk
    pc = chunk // PK
    lines_t = n // PK

    @functools.partial(
        pl.kernel,
        mesh=mesh,
        out_type=jax.ShapeDtypeStruct((NT * lines_t, 128), jnp.float32),
        scratch_types=[
            pltpu.VMEM((2, chunk), jnp.int32),
            pltpu.VMEM((2, PK, pc, DW), jnp.float32),
            pltpu.SemaphoreType.DMA,
            pltpu.SemaphoreType.DMA,
            pltpu.SemaphoreType.DMA,
            pltpu.SemaphoreType.DMA,
        ],
        compiler_params=pltpu.CompilerParams(use_tc_tiling_on_sc=False),
    )
    def gather_k(table_hbm, idx_hbm, out_hbm, idx_v, rows_v, g0, g1, w0, w1):
        wid = lax.axis_index("s") * 2 + lax.axis_index("c")
        gsem = [g0, g1]
        wsem = [w0, w1]
        chunks = [(t, g) for t in range(NT) for g in range(g_steps)]

        def load_idx(c, s):
            t, g = chunks[c]
            off = t * n + wid * tpw + g * chunk
            pltpu.sync_copy(idx_hbm.at[pl.ds(off, chunk)], idx_v.at[s])

        def start_gathers(s):
            return [
                pltpu.async_copy(
                    table_hbm.at[idx_v.at[s, pl.ds(k * pc, pc)]],
                    rows_v.at[s, k],
                    gsem[s])
                for k in range(PK)
            ]

        def start_wb(c, s):
            t, g = chunks[c]
            line0 = t * lines_t + wid * (tpw // PK) + g * pc
            return [
                pltpu.async_copy(
                    rows_v.at[s, k],
                    out_hbm.at[pl.ds(line0, pc), pl.ds(k * DW, DW)],
                    wsem[s])
                for k in range(PK)
            ]

        n_chunks = len(chunks)
        load_idx(0, 0)
        g_h = [start_gathers(0), None]
        w_h = [None, None]
        for c in range(n_chunks):
            s = c & 1
            o = s ^ 1
            if c + 1 < n_chunks:
                if w_h[o] is not None:
                    for h in w_h[o]:
                        h.wait()
                load_idx(c + 1, o)
                g_h[o] = start_gathers(o)
            for h in g_h[s]:
                h.wait()
            w_h[s] = start_wb(c, s)
        for hs in w_h:
            if hs is not None:
                for h in hs:
                    h.wait()

    return gather_k


def _make_fuse_body(pc, ll):
    bb = pc // ll
    def _fuse_body(x_ref, w2_ref, b2_ref, mm_ref, gam_ref, bet_ref, o_ref):
        w2 = w2_ref[...]
        b2 = b2_ref[...]
        mm = mm_ref[...]
        for k in range(PK):
            xk = jnp.concatenate(
                [x_ref[t][:, k * DW:(k + 1) * DW] for t in range(NT)],
                axis=1)
            lin = jnp.dot(xk, w2, preferred_element_type=jnp.float32) + b2
            gate = jax.nn.sigmoid(lin[:, :128] * 1.2)
            z = gate * lin[:, 128:]
            mu = jnp.dot(z, mm, preferred_element_type=jnp.float32)[:, 0:1]
            e2 = jnp.dot(z * z, mm, preferred_element_type=jnp.float32)[:, 0:1]
            rstd = lax.rsqrt(e2 - mu * mu + 1e-5)
            res = (z - mu) * rstd * gam_ref[...] + bet_ref[...]
            o_ref[pl.ds(k * bb, bb)] = res.reshape(bb, ll, 128)
    return _fuse_body


def kernel(event_type, fault_class, syscall_class, opcode_family,
           transition_type, result_class,
           W_event, W_fault, W_syscall, W_opcode, W_trans, W_result,
           gate_W, gate_b, trans_W, trans_b, ln_gamma, ln_beta):
    tables = [W_event, W_fault, W_syscall, W_opcode, W_trans, W_result]
    idxs = [event_type, fault_class, syscall_class, opcode_family,
            transition_type, result_class]
    widths = [t.shape[1] for t in tables]

    n = event_type.size
    chunk = n // NW
    pc = chunk // PK

    padded, shifted, row_base = [], [], 0
    for W, ix in zip(tables, idxs):
        padded.append(jnp.pad(W, ((0, 0), (0, DW - W.shape[1]))))
        shifted.append(ix.reshape(-1).astype(jnp.int32) + row_base)
        row_base += W.shape[0]
    big_table = jnp.concatenate(padded, axis=0)
    idx_all = jnp.concatenate(shifted)

    gathered = _gather_kernel(n, chunk)(big_table, idx_all)
    gathered = gathered.reshape(NT, n // PK, 128)

    def seg_w(W):
        out, r = [], 0
        for w in widths:
            out.append(jnp.pad(W[r:r + w], ((0, DW - w), (0, 0))))
            r += w
        return jnp.concatenate(out, axis=0)

    w2 = jnp.concatenate([seg_w(gate_W), seg_w(trans_W)], axis=1)
    b2 = jnp.concatenate([gate_b, trans_b]).reshape(1, 256)
    mm = jnp.zeros((128, 8), jnp.float32).at[:, 0].set(1.0 / 128.0)

    nb, ll = event_type.shape
    cb = chunk // ll

    out = pl.pallas_call(
        _make_fuse_body(pc, ll),
        grid=(n // chunk,),
        in_specs=[
            pl.BlockSpec((NT, pc, 128), lambda i: (0, i, 0)),
            pl.BlockSpec((NT * DW, 256), lambda i: (0, 0)),
            pl.BlockSpec((1, 256), lambda i: (0, 0)),
            pl.BlockSpec((128, 8), lambda i: (0, 0)),
            pl.BlockSpec((1, 128), lambda i: (0, 0)),
            pl.BlockSpec((1, 128), lambda i: (0, 0)),
        ],
        out_specs=pl.BlockSpec((cb, ll, 128), lambda i: (i, 0, 0)),
        out_shape=jax.ShapeDtypeStruct((nb, ll, 128), jnp.float32),
    )(gathered, w2, b2, mm,
      ln_gamma.reshape(1, 128), ln_beta.reshape(1, 128))

    return out


# speedup vs baseline: 2.3019x; 2.1819x over previous
import functools
import jax
import jax.numpy as jnp
from jax import lax
from jax.experimental import pallas as pl
from jax.experimental.pallas import tpu as pltpu
from jax.experimental.pallas import tpu_sc as plsc

DW = 8
PK = 128 // DW
NW = 32
NG = 5
RV = 8


def _gather_kernel(n, chunk):
    mesh = plsc.VectorSubcoreMesh(core_axis_name="c", subcore_axis_name="s")
    tpw = n // NW
    g_steps = tpw // chunk
    pc = chunk // PK
    lines_t = n // PK

    @functools.partial(
        pl.kernel,
        mesh=mesh,
        out_type=jax.ShapeDtypeStruct((NG * lines_t, 128), jnp.float32),
        scratch_types=[
            pltpu.VMEM((2, chunk), jnp.int32),
            pltpu.VMEM((2, PK, pc, DW), jnp.float32),
            pltpu.SemaphoreType.DMA,
            pltpu.SemaphoreType.DMA,
            pltpu.SemaphoreType.DMA,
            pltpu.SemaphoreType.DMA,
        ],
        compiler_params=pltpu.CompilerParams(use_tc_tiling_on_sc=False),
    )
    def gather_k(table_hbm, idx_hbm, out_hbm, idx_v, rows_v, g0, g1, w0, w1):
        wid = lax.axis_index("s") * 2 + lax.axis_index("c")
        gsem = [g0, g1]
        wsem = [w0, w1]
        chunks = [(t, g) for t in range(NG) for g in range(g_steps)]

        def load_idx(c, s):
            t, g = chunks[c]
            off = t * n + wid * tpw + g * chunk
            pltpu.sync_copy(idx_hbm.at[pl.ds(off, chunk)], idx_v.at[s])

        def start_gathers(s):
            return [
                pltpu.async_copy(
                    table_hbm.at[idx_v.at[s, pl.ds(k * pc, pc)]],
                    rows_v.at[s, k],
                    gsem[s])
                for k in range(PK)
            ]

        def start_wb(c, s):
            t, g = chunks[c]
            line0 = t * lines_t + wid * (tpw // PK) + g * pc
            return [
                pltpu.async_copy(
                    rows_v.at[s, k],
                    out_hbm.at[pl.ds(line0, pc), pl.ds(k * DW, DW)],
                    wsem[s])
                for k in range(PK)
            ]

        n_chunks = len(chunks)
        load_idx(0, 0)
        g_h = [start_gathers(0), None]
        w_h = [None, None]
        for c in range(n_chunks):
            s = c & 1
            o = s ^ 1
            if c + 1 < n_chunks:
                if w_h[o] is not None:
                    for h in w_h[o]:
                        h.wait()
                load_idx(c + 1, o)
                g_h[o] = start_gathers(o)
            for h in g_h[s]:
                h.wait()
            w_h[s] = start_wb(c, s)
        for hs in w_h:
            if hs is not None:
                for h in hs:
                    h.wait()

    return gather_k


def _make_fuse_body(pc, ll):
    bb = pc // ll
    def _fuse_body(x_ref, r_ref, w2_ref, b2_ref, mm_ref, gam_ref, bet_ref,
                   o_ref):
        w2 = w2_ref[...]
        b2 = b2_ref[...]
        mm = mm_ref[...]
        oh_iota = lax.broadcasted_iota(jnp.int32, (pc, RV), 1)
        for k in range(PK):
            onehot = (r_ref[0][:, k:k + 1] == oh_iota).astype(jnp.float32)
            xk = jnp.concatenate(
                [x_ref[t][:, k * DW:(k + 1) * DW] for t in range(NG)]
                + [onehot],
                axis=1)
            lin = jnp.dot(xk, w2, preferred_element_type=jnp.float32) + b2
            gate = jax.nn.sigmoid(lin[:, :128] * 1.2)
            z = gate * lin[:, 128:]
            mu = jnp.dot(z, mm, preferred_element_type=jnp.float32)[:, 0:1]
            e2 = jnp.dot(z * z, mm, preferred_element_type=jnp.float32)[:, 0:1]
            rstd = lax.rsqrt(e2 - mu * mu + 1e-5)
            res = (z - mu) * rstd * gam_ref[...] + bet_ref[...]
            o_ref[pl.ds(k * bb, bb)] = res.reshape(bb, ll, 128)
    return _fuse_body


def kernel(event_type, fault_class, syscall_class, opcode_family,
           transition_type, result_class,
           W_event, W_fault, W_syscall, W_opcode, W_trans, W_result,
           gate_W, gate_b, trans_W, trans_b, ln_gamma, ln_beta):
    tables = [W_event, W_fault, W_syscall, W_opcode, W_trans]
    idxs = [event_type, fault_class, syscall_class, opcode_family,
            transition_type]
    widths = [t.shape[1] for t in tables]

    n = event_type.size
    chunk = n // NW
    pc = chunk // PK

    padded, shifted, row_base = [], [], 0
    for W, ix in zip(tables, idxs):
        padded.append(jnp.pad(W, ((0, 0), (0, DW - W.shape[1]))))
        shifted.append(ix.reshape(-1).astype(jnp.int32) + row_base)
        row_base += W.shape[0]
    big_table = jnp.concatenate(padded, axis=0)
    idx_all = jnp.concatenate(shifted)

    gathered = _gather_kernel(n, chunk)(big_table, idx_all)
    gathered = gathered.reshape(NG, n // PK, 128)

    rp = (result_class.reshape(-1).astype(jnp.int32)
          .reshape(n // chunk, PK, pc).transpose(0, 2, 1))

    def seg_w(W):
        out, r = [], 0
        for w in widths:
            out.append(jnp.pad(W[r:r + w], ((0, DW - w), (0, 0))))
            r += w
        res = W_result @ W[r:r + W_result.shape[1]]
        out.append(res)
        return jnp.concatenate(out, axis=0)

    w2 = jnp.concatenate([seg_w(gate_W), seg_w(trans_W)], axis=1)
    b2 = jnp.concatenate([gate_b, trans_b]).reshape(1, 256)
    mm = jnp.zeros((128, 8), jnp.float32).at[:, 0].set(1.0 / 128.0)

    nb, ll = event_type.shape
    cb = chunk // ll

    out = pl.pallas_call(
        _make_fuse_body(pc, ll),
        grid=(n // chunk,),
        in_specs=[
            pl.BlockSpec((NG, pc, 128), lambda i: (0, i, 0)),
            pl.BlockSpec((1, pc, PK), lambda i: (i, 0, 0)),
            pl.BlockSpec((NG * DW + RV, 256), lambda i: (0, 0)),
            pl.BlockSpec((1, 256), lambda i: (0, 0)),
            pl.BlockSpec((128, 8), lambda i: (0, 0)),
            pl.BlockSpec((1, 128), lambda i: (0, 0)),
            pl.BlockSpec((1, 128), lambda i: (0, 0)),
        ],
        out_specs=pl.BlockSpec((cb, ll, 128), lambda i: (i, 0, 0)),
        out_shape=jax.ShapeDtypeStruct((nb, ll, 128), jnp.float32),
    )(gathered, rp, w2, b2, mm,
      ln_gamma.reshape(1, 128), ln_beta.reshape(1, 128))

    return out
